# Initial kernel scaffold; baseline (speedup 1.0000x reference)
import jax
import jax.numpy as jnp
from jax.experimental import pallas as pl


def kernel(word_seq, embedding_table):
    raise NotImplementedError("write your pallas kernel here")


# speedup vs baseline: 8.2156x; 8.2156x over previous
import jax
import jax.numpy as jnp
from jax.experimental import pallas as pl

_BLOCK = 4096
_DIM = 512
_ROWS_PAD = 16


def _emb_block(idx_ref, tab_ref, out_ref):
    idx = idx_ref[0, 0, :]
    onehot = (
        idx[:, None] == jax.lax.broadcasted_iota(jnp.int32, (1, _ROWS_PAD), 1)
    ).astype(jnp.float32)
    out_ref[...] = jnp.dot(
        onehot, tab_ref[...], preferred_element_type=jnp.float32
    )


def kernel(word_seq, embedding_table):
    s0, s1 = word_seq.shape
    n = s0 * s1
    num_rows, dim = embedding_table.shape
    idx = word_seq.reshape(n).astype(jnp.int32)
    grid = n // _BLOCK
    idx3 = idx.reshape(grid, 1, _BLOCK)
    tab = jnp.pad(embedding_table, ((0, _ROWS_PAD - num_rows), (0, 0)))
    out = pl.pallas_call(
        _emb_block,
        grid=(grid,),
        in_specs=[
            pl.BlockSpec((1, 1, _BLOCK), lambda i: (i, 0, 0)),
            pl.BlockSpec((_ROWS_PAD, _DIM), lambda i: (0, 0)),
        ],
        out_specs=pl.BlockSpec((_BLOCK, _DIM), lambda i: (i, 0)),
        out_shape=jax.ShapeDtypeStruct((n, _DIM), jnp.float32),
    )(idx3, tab)
    return out.reshape(s0, s1, dim)
